# Initial kernel scaffold; baseline (speedup 1.0000x reference)
import jax
import jax.numpy as jnp
from jax.experimental import pallas as pl


def kernel(feat, edge_index, edge_type, W_e1, b_e1, W_e2, b_e2, edge_table, Wl1, bl1, Wl2, bl2, eps):
    raise NotImplementedError("write your pallas kernel here")


# speedup vs baseline: 1.2269x; 1.2269x over previous
import functools

import jax
import jax.numpy as jnp
from jax import lax
from jax.experimental import pallas as pl
from jax.experimental.pallas import tpu as pltpu
from jax.experimental.pallas import tpu_sc as plsc

N = 10000
E = 320000
H = 128
L = 4
N_EDGE_TYPES = 16

NC = 2
NS = 16
NW = NC * NS
CHUNK = 128
G = 80
EPW = G * CHUNK
E_PAD = NW * EPW
RPT = 632
N_ACC = NS * RPT
DUMMY = N


def _mp_body(h_hbm, table_hbm, src_hbm, type_hbm, dst_hbm, out_hbm,
             hbuf, ebuf, zbuf, sbuf, tbuf, dbuf, agg_sh, sem_h, sem_e):
    cid = lax.axis_index("c")
    sid = lax.axis_index("s")
    wid = sid * NC + cid

    zero16 = jnp.zeros((16,), jnp.float32)

    @pl.loop(0, 128)
    def _zrow(i):
        for j in range(8):
            zbuf[i, pl.ds(j * 16, 16)] = zero16

    base = sid * RPT
    for k in range(4):
        pltpu.sync_copy(zbuf, agg_sh.at[pl.ds(base + k * 128, 128)])
    pltpu.sync_copy(zbuf.at[pl.ds(0, RPT - 512)],
                    agg_sh.at[pl.ds(base + 512, RPT - 512)])
    plsc.subcore_barrier()

    ebase = wid * EPW

    @pl.loop(0, G)
    def _chunk(g):
        off = ebase + g * CHUNK
        pltpu.sync_copy(src_hbm.at[pl.ds(off, CHUNK)], sbuf)
        pltpu.sync_copy(type_hbm.at[pl.ds(off, CHUNK)], tbuf)
        pltpu.sync_copy(dst_hbm.at[pl.ds(off, CHUNK)], dbuf)
        cp_e = pltpu.async_copy(table_hbm.at[tbuf], ebuf, sem_e)
        cp_h = pltpu.async_copy(h_hbm.at[sbuf], hbuf, sem_h)
        cp_e.wait()
        cp_h.wait()

        @pl.loop(0, CHUNK)
        def _row(i):
            for j in range(8):
                s = pl.ds(j * 16, 16)
                hbuf[i, s] = jnp.maximum(hbuf[i, s] + ebuf[i, s], 0.0)

        pltpu.sync_copy(hbuf, agg_sh.at[dbuf], add=True)

    plsc.subcore_barrier()

    for k in range(4):
        r = pl.ds(base + k * 128, 128)
        pltpu.sync_copy(agg_sh.at[r], hbuf)
        pltpu.sync_copy(hbuf, out_hbm.at[cid, r])
    rtail = pl.ds(base + 512, RPT - 512)
    pltpu.sync_copy(agg_sh.at[rtail], hbuf.at[pl.ds(0, RPT - 512)])
    pltpu.sync_copy(hbuf.at[pl.ds(0, RPT - 512)], out_hbm.at[cid, rtail])


_mp = functools.partial(
    pl.kernel,
    out_type=jax.ShapeDtypeStruct((NC, N_ACC, H), jnp.float32),
    mesh=plsc.VectorSubcoreMesh(core_axis_name="c", subcore_axis_name="s"),
    scratch_types=[
        pltpu.VMEM((CHUNK, H), jnp.float32),
        pltpu.VMEM((CHUNK, H), jnp.float32),
        pltpu.VMEM((128, H), jnp.float32),
        pltpu.VMEM((CHUNK,), jnp.int32),
        pltpu.VMEM((CHUNK,), jnp.int32),
        pltpu.VMEM((CHUNK,), jnp.int32),
        pltpu.VMEM_SHARED((N_ACC, H), jnp.float32),
        pltpu.SemaphoreType.DMA,
        pltpu.SemaphoreType.DMA,
    ],
)(_mp_body)


BR = 1000
NB = N // BR


def _embed_body(x_ref, w1_ref, b1_ref, w2_ref, b2_ref, o_ref):
    z = jnp.maximum(
        jnp.dot(x_ref[...], w1_ref[...], preferred_element_type=jnp.float32)
        + b1_ref[...], 0.0)
    o_ref[...] = (
        jnp.dot(z, w2_ref[...], preferred_element_type=jnp.float32)
        + b2_ref[...])


def _embed(feat, w1, b1, w2, b2):
    full = pl.BlockSpec((H, H), lambda i: (0, 0))
    vec = pl.BlockSpec((1, H), lambda i: (0, 0))
    return pl.pallas_call(
        _embed_body,
        grid=(NB,),
        in_specs=[pl.BlockSpec((BR, H), lambda i: (i, 0)), full, vec, full, vec],
        out_specs=pl.BlockSpec((BR, H), lambda i: (i, 0)),
        out_shape=jax.ShapeDtypeStruct((N, H), jnp.float32),
    )(feat, w1, b1.reshape(1, H), w2, b2.reshape(1, H))


def _layer_body(h_ref, agg_ref, w1_ref, b1_ref, w2_ref, b2_ref, eps_ref, o_ref):
    eps = eps_ref[0, 0]
    hv = h_ref[...]
    z = (1.0 + eps) * hv + agg_ref[0] + agg_ref[1]
    z = jnp.maximum(
        jnp.dot(z, w1_ref[...], preferred_element_type=jnp.float32)
        + b1_ref[...], 0.0)
    z = jnp.dot(z, w2_ref[...], preferred_element_type=jnp.float32) + b2_ref[...]
    o_ref[...] = jnp.maximum(z, 0.0) + hv


def _layer(h, agg2, w1, b1, w2, b2, eps_l):
    full = pl.BlockSpec((H, H), lambda i: (0, 0))
    vec = pl.BlockSpec((1, H), lambda i: (0, 0))
    scal = pl.BlockSpec(memory_space=pltpu.SMEM)
    return pl.pallas_call(
        _layer_body,
        grid=(NB,),
        in_specs=[
            pl.BlockSpec((BR, H), lambda i: (i, 0)),
            pl.BlockSpec((NC, BR, H), lambda i: (0, i, 0)),
            full, vec, full, vec, scal,
        ],
        out_specs=pl.BlockSpec((BR, H), lambda i: (i, 0)),
        out_shape=jax.ShapeDtypeStruct((N, H), jnp.float32),
    )(h, agg2, w1, b1.reshape(1, H), w2, b2.reshape(1, H),
      eps_l.reshape(1, 1))


NF = 6 * H


def _final_body(f0, f1, f2, f3, f4, f5, o_ref, macc):
    p = pl.program_id(0)
    b = pl.program_id(1)
    cat = jnp.concatenate(
        [f0[...], f1[...], f2[...], f3[...], f4[...], f5[...]], axis=1)

    @pl.when(p == 0)
    def _():
        m = jnp.max(cat, axis=0, keepdims=True)
        m8 = jnp.broadcast_to(m, (8, NF))

        @pl.when(b == 0)
        def _():
            macc[...] = m8

        @pl.when(b > 0)
        def _():
            macc[...] = jnp.maximum(macc[...], m8)

    @pl.when(p == 1)
    def _():
        o_ref[:, :NF] = cat
        o_ref[:, NF:] = jnp.broadcast_to(macc[0:1, :], (BR, NF))


def _final(feats):
    blk = pl.BlockSpec((BR, H), lambda p, b: (b, 0))
    return pl.pallas_call(
        _final_body,
        grid=(2, NB),
        in_specs=[blk] * 6,
        out_specs=pl.BlockSpec((BR, 2 * NF), lambda p, b: (b, 0)),
        out_shape=jax.ShapeDtypeStruct((N, 2 * NF), jnp.float32),
        scratch_shapes=[pltpu.VMEM((8, NF), jnp.float32)],
    )(*feats)


def kernel(feat, edge_index, edge_type, W_e1, b_e1, W_e2, b_e2,
           edge_table, Wl1, bl1, Wl2, bl2, eps):
    pad = E_PAD - E
    src = jnp.concatenate([edge_index[0], jnp.zeros((pad,), jnp.int32)])
    dst = jnp.concatenate([edge_index[1], jnp.full((pad,), DUMMY, jnp.int32)])
    ety = jnp.concatenate([edge_type, jnp.zeros((pad,), jnp.int32)])

    h = _embed(feat, W_e1, b_e1, W_e2, b_e2)
    feats = [feat, h]
    for l in range(L):
        agg2 = _mp(h, edge_table, src, ety, dst)
        h = _layer(h, agg2, Wl1[l], bl1[l], Wl2[l], bl2[l], eps[l])
        feats.append(h)
    return _final(feats)
